# baseline (speedup 1.0000x reference)
import functools

import jax
import jax.numpy as jnp
from jax import lax
from jax.experimental import pallas as pl
from jax.experimental.pallas import tpu as pltpu
from jax.experimental.pallas import tpu_sc as plsc

N_NODES = 10000
N_EDGES = 320000
D_FEAT = 128
DW = D_FEAT // 2

NC = 2
NS = 16
NW = NC * NS
EPW = N_EDGES // NW
C = 80
NCHUNK = EPW // C
LANES = 16


def _edge_decoder_kernel(zs_hbm, zm_hbm, row_hbm, col_hbm, out_hbm,
                         rowi_v, coli_v, outw_v,
                         src_a, dst_a, src_b, dst_b, sem_a, sem_b):
    wid = lax.axis_index("s") * NC + lax.axis_index("c")
    base = wid * EPW
    pltpu.sync_copy(row_hbm.at[pl.ds(base, EPW)], rowi_v)
    pltpu.sync_copy(col_hbm.at[pl.ds(base, EPW)], coli_v)

    def issue(g, src_v, dst_v, sem):
        pltpu.async_copy(zs_hbm.at[rowi_v.at[pl.ds(g * C, C)]], src_v, sem)
        pltpu.async_copy(zm_hbm.at[coli_v.at[pl.ds(g * C, C)]], dst_v, sem)

    def wait(src_v, dst_v, sem):
        pltpu.make_async_copy(zs_hbm.at[rowi_v.at[pl.ds(0, C)]],
                              src_v, sem).wait()
        pltpu.make_async_copy(zm_hbm.at[coli_v.at[pl.ds(0, C)]],
                              dst_v, sem).wait()

    lane = lax.iota(jnp.int32, LANES)

    def compute(g, src_v, dst_v):
        def edge16(i, c2):
            lane_e = lane + i * LANES
            acc0 = jnp.zeros((LANES,), jnp.float32)
            acc1 = jnp.zeros((LANES,), jnp.float32)
            acc2 = jnp.zeros((LANES,), jnp.float32)
            acc3 = jnp.zeros((LANES,), jnp.float32)
            for k in range(DW):
                fk = jnp.full((LANES,), k, jnp.int32)
                s = plsc.bitcast(plsc.load_gather(src_v, [lane_e, fk]),
                                 jnp.bfloat16)
                d = plsc.bitcast(plsc.load_gather(dst_v, [lane_e, fk]),
                                 jnp.bfloat16)
                qa, qb = plsc.unpack(s * d,
                                     format=plsc.PackFormat.INTERLEAVED)
                if k % 2 == 0:
                    acc0 = acc0 + qa
                    acc1 = acc1 + qb
                else:
                    acc2 = acc2 + qa
                    acc3 = acc3 + qb
            outw_v[pl.ds(g * C + i * LANES, LANES)] = (acc0 + acc1) + (acc2 + acc3)
            return c2

        lax.fori_loop(0, C // LANES, edge16, 0, unroll=False)

    issue(0, src_a, dst_a, sem_a)

    def pair_body(h, carry):
        g0 = 2 * h
        issue(g0 + 1, src_b, dst_b, sem_b)
        wait(src_a, dst_a, sem_a)
        compute(g0, src_a, dst_a)
        issue(g0 + 2, src_a, dst_a, sem_a)
        wait(src_b, dst_b, sem_b)
        compute(g0 + 1, src_b, dst_b)
        return carry

    lax.fori_loop(0, (NCHUNK - 1) // 2, pair_body, 0, unroll=False)
    wait(src_a, dst_a, sem_a)
    compute(NCHUNK - 1, src_a, dst_a)

    pltpu.sync_copy(outw_v, out_hbm.at[pl.ds(base, EPW)])


def _pack_table(z):
    z16 = z.astype(jnp.bfloat16).reshape(N_NODES, DW, 2)
    return lax.bitcast_convert_type(z16, jnp.int32)


@jax.jit
def kernel(z_srna, z_mrna, edge_label_index):
    row = edge_label_index[0]
    col = edge_label_index[1]
    zs_p = _pack_table(z_srna)
    zm_p = _pack_table(z_mrna)
    mesh = plsc.VectorSubcoreMesh(core_axis_name="c", subcore_axis_name="s")
    f = pl.kernel(
        _edge_decoder_kernel,
        mesh=mesh,
        out_type=jax.ShapeDtypeStruct((N_EDGES,), jnp.float32),
        compiler_params=pltpu.CompilerParams(needs_layout_passes=False,
                                             use_tc_tiling_on_sc=False),
        scratch_types=[
            pltpu.VMEM((EPW,), jnp.int32),
            pltpu.VMEM((EPW,), jnp.int32),
            pltpu.VMEM((EPW,), jnp.float32),
            pltpu.VMEM((C, DW), jnp.int32),
            pltpu.VMEM((C, DW), jnp.int32),
            pltpu.VMEM((C, DW), jnp.int32),
            pltpu.VMEM((C, DW), jnp.int32),
            pltpu.SemaphoreType.DMA,
            pltpu.SemaphoreType.DMA,
        ],
    )
    return f(zs_p, zm_p, row, col)


# speedup vs baseline: 4.6164x; 4.6164x over previous
import functools

import jax
import jax.numpy as jnp
from jax import lax
from jax.experimental import pallas as pl
from jax.experimental.pallas import tpu as pltpu
from jax.experimental.pallas import tpu_sc as plsc

N_NODES = 10000
N_EDGES = 320000
D_FEAT = 128
DW = D_FEAT // 2

NC = 2
NS = 16
NW = NC * NS
EPW = N_EDGES // NW
C = 80
NCHUNK = EPW // C
LANES = 16


def _edge_decoder_kernel(zs_hbm, zm_hbm, row_hbm, col_hbm, out_hbm,
                         shr_s, shr_m,
                         rowi_v, coli_v, outw_v,
                         src_a, dst_a, src_b, dst_b, sem_a, sem_b):
    sid = lax.axis_index("s")
    wid = sid * NC + lax.axis_index("c")
    base = wid * EPW

    rows_per_sub = N_NODES // NS
    sbase = sid * rows_per_sub
    pltpu.sync_copy(zs_hbm.at[pl.ds(sbase, rows_per_sub)],
                    shr_s.at[pl.ds(sbase, rows_per_sub)])
    pltpu.sync_copy(zm_hbm.at[pl.ds(sbase, rows_per_sub)],
                    shr_m.at[pl.ds(sbase, rows_per_sub)])
    pltpu.sync_copy(row_hbm.at[pl.ds(base, EPW)], rowi_v)
    pltpu.sync_copy(col_hbm.at[pl.ds(base, EPW)], coli_v)
    plsc.subcore_barrier()

    def issue(g, src_v, dst_v, sem):
        pltpu.async_copy(shr_s.at[rowi_v.at[pl.ds(g * C, C)]], src_v, sem)
        pltpu.async_copy(shr_m.at[coli_v.at[pl.ds(g * C, C)]], dst_v, sem)

    def wait(src_v, dst_v, sem):
        pltpu.make_async_copy(shr_s.at[rowi_v.at[pl.ds(0, C)]],
                              src_v, sem).wait()
        pltpu.make_async_copy(shr_m.at[coli_v.at[pl.ds(0, C)]],
                              dst_v, sem).wait()

    lane = lax.iota(jnp.int32, LANES)
    masks = [lane == j for j in range(LANES)]

    def compute(g, src_v, dst_v):
        def edge16(i, c2):
            acc = jnp.zeros((LANES,), jnp.float32)
            for j in range(LANES):
                e = i * LANES + j
                p = jnp.zeros((LANES,), jnp.float32)
                for k in range(DW // LANES):
                    s = plsc.bitcast(src_v[e, pl.ds(k * LANES, LANES)],
                                     jnp.bfloat16)
                    d = plsc.bitcast(dst_v[e, pl.ds(k * LANES, LANES)],
                                     jnp.bfloat16)
                    qa, qb = plsc.unpack(s * d,
                                         format=plsc.PackFormat.INTERLEAVED)
                    p = p + qa + qb
                acc = jnp.where(masks[j], jnp.sum(p), acc)
            outw_v[pl.ds(g * C + i * LANES, LANES)] = acc
            return c2

        lax.fori_loop(0, C // LANES, edge16, 0, unroll=False)

    issue(0, src_a, dst_a, sem_a)

    def pair_body(h, carry):
        g0 = 2 * h
        issue(g0 + 1, src_b, dst_b, sem_b)
        wait(src_a, dst_a, sem_a)
        compute(g0, src_a, dst_a)
        issue(g0 + 2, src_a, dst_a, sem_a)
        wait(src_b, dst_b, sem_b)
        compute(g0 + 1, src_b, dst_b)
        return carry

    lax.fori_loop(0, (NCHUNK - 1) // 2, pair_body, 0, unroll=False)
    wait(src_a, dst_a, sem_a)
    compute(NCHUNK - 1, src_a, dst_a)

    pltpu.sync_copy(outw_v, out_hbm.at[pl.ds(base, EPW)])


def _pack_table(z):
    z16 = z.astype(jnp.bfloat16).reshape(N_NODES, DW, 2)
    return lax.bitcast_convert_type(z16, jnp.int32)


@jax.jit
def kernel(z_srna, z_mrna, edge_label_index):
    row = edge_label_index[0]
    col = edge_label_index[1]
    zs_p = _pack_table(z_srna)
    zm_p = _pack_table(z_mrna)
    mesh = plsc.VectorSubcoreMesh(core_axis_name="c", subcore_axis_name="s")
    f = pl.kernel(
        _edge_decoder_kernel,
        mesh=mesh,
        out_type=jax.ShapeDtypeStruct((N_EDGES,), jnp.float32),
        compiler_params=pltpu.CompilerParams(needs_layout_passes=False,
                                             use_tc_tiling_on_sc=False),
        scratch_types=[
            pltpu.VMEM_SHARED((N_NODES, DW), jnp.int32),
            pltpu.VMEM_SHARED((N_NODES, DW), jnp.int32),
            pltpu.VMEM((EPW,), jnp.int32),
            pltpu.VMEM((EPW,), jnp.int32),
            pltpu.VMEM((EPW,), jnp.float32),
            pltpu.VMEM((C, DW), jnp.int32),
            pltpu.VMEM((C, DW), jnp.int32),
            pltpu.VMEM((C, DW), jnp.int32),
            pltpu.VMEM((C, DW), jnp.int32),
            pltpu.SemaphoreType.DMA,
            pltpu.SemaphoreType.DMA,
        ],
    )
    return f(zs_p, zm_p, row, col)
